# Initial kernel scaffold; baseline (speedup 1.0000x reference)
import jax
import jax.numpy as jnp
from jax.experimental import pallas as pl


def kernel(x_a, edge_index_a, batch_a, x_t, edge_index_t, batch_t, W1, b1, W2, b2, W3, b3, Wg1, bg1, Wg2, bg2, Wf1, bf1, Wf2, bf2, Wo, bo):
    raise NotImplementedError("write your pallas kernel here")


# speedup vs baseline: 9.4624x; 9.4624x over previous
import functools

import jax
import jax.numpy as jnp
from jax import lax
from jax.experimental import pallas as pl
from jax.experimental.pallas import tpu as pltpu
from jax.experimental.pallas import tpu_sc as plsc

f32 = jnp.float32
i32 = jnp.int32

N = 10000
NPAD = 10240
E = 320000
B = 128
D = 128

NC = 2
NS = 16
NW = NC * NS
EPW = E // NW
EC = 80
NCHUNK = EPW // EC
STRIPE = NPAD // NS

RBASE = 312
RPT = 352
PCH = 32
SEGS = 128

@functools.cache
def _mesh():
    return plsc.VectorSubcoreMesh(core_axis_name="c", subcore_axis_name="s",
                                  num_cores=NC, num_subcores=NS)


@functools.cache
def _deg_kernel():
    return pl.kernel(
        _deg_body,
        out_type=jax.ShapeDtypeStruct((2 * NW * NPAD,), f32),
        mesh=_mesh(),
        compiler_params=pltpu.CompilerParams(needs_layout_passes=False),
        scratch_types=[
            pltpu.VMEM((EPW,), i32),
            pltpu.VMEM((NPAD,), f32),
        ],
    )


def _deg_body(dst_a, dst_t, out, dstv, acc):
    c = lax.axis_index("c")
    s = lax.axis_index("s")
    wid = c * NS + s
    for br, dref in ((0, dst_a), (1, dst_t)):
        def zb(i, _):
            acc[pl.ds(i * 16, 16)] = jnp.zeros((16,), f32)
            return 0

        lax.fori_loop(0, NPAD // 16, zb, 0)
        pltpu.sync_copy(dref.at[pl.ds(wid * EPW, EPW)], dstv)

        def body(k, _):
            idx = dstv[pl.ds(k * 16, 16)]
            plsc.addupdate_scatter(acc, [idx], jnp.ones((16,), f32))
            return 0

        lax.fori_loop(0, EPW // 16, body, 0)
        pltpu.sync_copy(acc, out.at[pl.ds((br * NW + wid) * NPAD, NPAD)])


@functools.cache
def _segsum_kernel():
    return pl.kernel(
        _segsum_body,
        out_type=jax.ShapeDtypeStruct((NC, NPAD, D), f32),
        mesh=_mesh(),
        scratch_types=[
            pltpu.VMEM((EC,), i32),
            pltpu.VMEM((EC,), i32),
            pltpu.VMEM((EC, D), f32),
            pltpu.VMEM((128, D), f32),
            pltpu.VMEM_SHARED((NPAD, D), f32),
            pltpu.SemaphoreType.DMA,
        ],
    )


def _segsum_body(table, src_hbm, dst_hbm, zeros_d, out, srcv, dstv, rows,
                 stage, acc, sem):
    c = lax.axis_index("c")
    s = lax.axis_index("s")
    wid = c * NS + s
    pltpu.sync_copy(zeros_d, stage)
    for j in range(STRIPE // 128):
        pltpu.sync_copy(stage, acc.at[pl.ds(s * STRIPE + j * 128, 128)])
    plsc.subcore_barrier()

    def body(k, _):
        off = wid * EPW + k * EC
        pltpu.sync_copy(src_hbm.at[pl.ds(off, EC)], srcv)
        pltpu.sync_copy(dst_hbm.at[pl.ds(off, EC)], dstv)
        pltpu.async_copy(table.at[srcv], rows, sem).wait()
        pltpu.sync_copy(rows, acc.at[dstv], add=True)
        return 0

    lax.fori_loop(0, NCHUNK, body, 0)
    plsc.subcore_barrier()
    for j in range(STRIPE // 128):
        pltpu.sync_copy(acc.at[pl.ds(s * STRIPE + j * 128, 128)], stage)

        @pl.when(c == 0)
        def _():
            pltpu.sync_copy(stage, out.at[0, pl.ds(s * STRIPE + j * 128, 128)])

        @pl.when(c == 1)
        def _():
            pltpu.sync_copy(stage, out.at[1, pl.ds(s * STRIPE + j * 128, 128)])


@functools.cache
def _pool_kernel():
    return pl.kernel(
        _pool_body,
        out_type=jax.ShapeDtypeStruct((NW * SEGS * 512,), f32),
        mesh=_mesh(),
        compiler_params=pltpu.CompilerParams(needs_layout_passes=False),
        scratch_types=[
            pltpu.VMEM((RPT,), i32),
            pltpu.VMEM((PCH * 512,), f32),
            pltpu.VMEM((129 * 512,), f32),
        ],
    )


def _pool_body(h3_flat, batch_pad, out, bv, rbuf, acc):
    c = lax.axis_index("c")
    s = lax.axis_index("s")
    wid = c * NS + s
    base = wid * RBASE

    def zb(i, _):
        acc[pl.ds(i * 16, 16)] = jnp.full((16,), -1.0, f32)
        return 0

    lax.fori_loop(0, (129 * 512) // 16, zb, 0)
    pltpu.sync_copy(batch_pad.at[pl.ds(base, RPT)], bv)
    lane = lax.iota(i32, 16)

    def chunk(j, _):
        pltpu.sync_copy(h3_flat.at[pl.ds((base + j * PCH) * 512, PCH * 512)],
                        rbuf)

        def row(l, _):
            bidx = jnp.full((16,), 0, i32) + (j * PCH + l)
            b16 = plsc.load_gather(bv, [bidx])
            addr0 = b16 * 512 + lane
            roff = l * 512
            for q in range(32):
                addr = addr0 + q * 16
                a = plsc.load_gather(acc, [addr])
                r = rbuf[pl.ds(roff + q * 16, 16)]
                plsc.store_scatter(acc, [addr], jnp.maximum(a, r))
            return 0

        lax.fori_loop(0, PCH, row, 0)
        return 0

    lax.fori_loop(0, RPT // PCH, chunk, 0)
    pltpu.sync_copy(acc.at[pl.ds(0, SEGS * 512)],
                    out.at[pl.ds(wid * SEGS * 512, SEGS * 512)])


BLK = 1024


def _tc_pre(x_pad, deg_partsT):
    def body(x_ref, d_ref, xp_ref, dinv_ref):
        deg = jnp.sum(d_ref[...], axis=1, keepdims=True)
        dinv = lax.rsqrt(deg + 1.0)
        xp_ref[...] = x_ref[...] * dinv
        dinv_ref[...] = dinv

    return pl.pallas_call(
        body,
        grid=(NPAD // BLK,),
        in_specs=[pl.BlockSpec((BLK, D), lambda i: (i, 0)),
                  pl.BlockSpec((BLK, NW), lambda i: (i, 0))],
        out_specs=[pl.BlockSpec((BLK, D), lambda i: (i, 0)),
                   pl.BlockSpec((BLK, 1), lambda i: (i, 0))],
        out_shape=[jax.ShapeDtypeStruct((NPAD, D), f32),
                   jax.ShapeDtypeStruct((NPAD, 1), f32)],
    )(x_pad, deg_partsT)


def _tc_layer1(parts, xp, dinv, W, b):
    def body(p_ref, x_ref, v_ref, w_ref, b_ref, o_ref):
        p = p_ref[...]
        dv = v_ref[...]
        t = (p[0] + p[1] + x_ref[...]) * dv
        h = jnp.dot(t, w_ref[...], preferred_element_type=f32) + b_ref[...]
        o_ref[...] = jnp.maximum(h, 0.0) * dv

    return pl.pallas_call(
        body,
        grid=(NPAD // BLK,),
        in_specs=[pl.BlockSpec((NC, BLK, D), lambda i: (0, i, 0)),
                  pl.BlockSpec((BLK, D), lambda i: (i, 0)),
                  pl.BlockSpec((BLK, 1), lambda i: (i, 0)),
                  pl.BlockSpec((D, D), lambda i: (0, 0)),
                  pl.BlockSpec((1, D), lambda i: (0, 0))],
        out_specs=pl.BlockSpec((BLK, D), lambda i: (i, 0)),
        out_shape=jax.ShapeDtypeStruct((NPAD, D), f32),
    )(parts, xp, dinv, W, b)


def _tc_layer2(parts, h1p, dinv, W, b):
    def body(p_ref, x_ref, v_ref, w_ref, b_ref, o0_ref, o1_ref):
        p = p_ref[...]
        dv = v_ref[...]
        t = (p[0] + p[1] + x_ref[...]) * dv
        h = jnp.dot(t, w_ref[...], preferred_element_type=f32) + b_ref[...]
        hp = jnp.maximum(h, 0.0) * dv
        o0_ref[...] = hp[:, :D]
        o1_ref[...] = hp[:, D:]

    return pl.pallas_call(
        body,
        grid=(NPAD // BLK,),
        in_specs=[pl.BlockSpec((NC, BLK, D), lambda i: (0, i, 0)),
                  pl.BlockSpec((BLK, D), lambda i: (i, 0)),
                  pl.BlockSpec((BLK, 1), lambda i: (i, 0)),
                  pl.BlockSpec((D, 2 * D), lambda i: (0, 0)),
                  pl.BlockSpec((1, 2 * D), lambda i: (0, 0))],
        out_specs=[pl.BlockSpec((BLK, D), lambda i: (i, 0)),
                   pl.BlockSpec((BLK, D), lambda i: (i, 0))],
        out_shape=[jax.ShapeDtypeStruct((NPAD, D), f32),
                   jax.ShapeDtypeStruct((NPAD, D), f32)],
    )(parts, h1p, dinv, W, b)


def _tc_layer3(pa, pb, h2p0, h2p1, dinv, W, b):
    def body(pa_ref, pb_ref, x0_ref, x1_ref, v_ref, w_ref, b_ref, o_ref):
        pa_ = pa_ref[...]
        pb_ = pb_ref[...]
        dv = v_ref[...]
        t0 = (pa_[0] + pa_[1] + x0_ref[...]) * dv
        t1 = (pb_[0] + pb_[1] + x1_ref[...]) * dv
        t = jnp.concatenate([t0, t1], axis=1)
        h = jnp.dot(t, w_ref[...], preferred_element_type=f32) + b_ref[...]
        o_ref[...] = jnp.maximum(h, 0.0)

    return pl.pallas_call(
        body,
        grid=(NPAD // BLK,),
        in_specs=[pl.BlockSpec((NC, BLK, D), lambda i: (0, i, 0)),
                  pl.BlockSpec((NC, BLK, D), lambda i: (0, i, 0)),
                  pl.BlockSpec((BLK, D), lambda i: (i, 0)),
                  pl.BlockSpec((BLK, D), lambda i: (i, 0)),
                  pl.BlockSpec((BLK, 1), lambda i: (i, 0)),
                  pl.BlockSpec((2 * D, 4 * D), lambda i: (0, 0)),
                  pl.BlockSpec((1, 4 * D), lambda i: (0, 0))],
        out_specs=pl.BlockSpec((BLK, 4 * D), lambda i: (i, 0)),
        out_shape=jax.ShapeDtypeStruct((NPAD, 4 * D), f32),
    )(pa, pb, h2p0, h2p1, dinv, W, b)


def _tc_head(pa, pt, Wg1, bg1, Wg2, bg2, Wf1, bf1, Wf2, bf2, Wo_p, bo_p):
    def body(pa_ref, pt_ref, wg1, bg1r, wg2, bg2r, wf1, bf1r, wf2, bf2r, wo,
             bor, o_ref):
        def gbranch(pref):
            pm = jnp.max(pref[...], axis=0)
            pm = jnp.where(pm < 0.0, 0.0, pm)
            g = jnp.dot(pm, wg1[...], preferred_element_type=f32) + bg1r[...]
            g = jnp.maximum(g, 0.0)
            return jnp.dot(g, wg2[...], preferred_element_type=f32) + bg2r[...]

        ga = gbranch(pa_ref)
        gt = gbranch(pt_ref)
        xc = jnp.concatenate([ga, gt], axis=1)
        y = jnp.dot(xc, wf1[...], preferred_element_type=f32) + bf1r[...]
        y = jnp.maximum(y, 0.0)
        y = jnp.dot(y, wf2[...], preferred_element_type=f32) + bf2r[...]
        y = jnp.maximum(y, 0.0)
        o_ref[...] = jnp.dot(y, wo[...], preferred_element_type=f32) + bor[...]

    return pl.pallas_call(
        body,
        out_shape=jax.ShapeDtypeStruct((B, 128), f32),
    )(pa, pt, Wg1, bg1, Wg2, bg2, Wf1, bf1, Wf2, bf2, Wo_p, bo_p)


def _branch(x, src, dst, batch_pad, deg_partsT, zeros_d, W1, b1, W2, b2, W3,
            b3):
    x_pad = jnp.pad(x, ((0, NPAD - N), (0, 0)))
    xp, dinv = _tc_pre(x_pad, deg_partsT)
    p1 = _segsum_kernel()(xp, src, dst, zeros_d)
    h1p = _tc_layer1(p1, xp, dinv, W1, b1)
    p2 = _segsum_kernel()(h1p, src, dst, zeros_d)
    h2p0, h2p1 = _tc_layer2(p2, h1p, dinv, W2, b2)
    pa = _segsum_kernel()(h2p0, src, dst, zeros_d)
    pb = _segsum_kernel()(h2p1, src, dst, zeros_d)
    h3 = _tc_layer3(pa, pb, h2p0, h2p1, dinv, W3, b3)
    pool = _pool_kernel()(h3.reshape(-1), batch_pad)
    return pool.reshape(NW, SEGS, 512)


def kernel(x_a, edge_index_a, batch_a, x_t, edge_index_t, batch_t,
           W1, b1, W2, b2, W3, b3, Wg1, bg1, Wg2, bg2,
           Wf1, bf1, Wf2, bf2, Wo, bo):
    src_a = edge_index_a[0]
    dst_a = edge_index_a[1]
    src_t = edge_index_t[0]
    dst_t = edge_index_t[1]
    pad_b = jnp.full((NPAD - N,), SEGS, i32)
    batch_a_pad = jnp.concatenate([batch_a, pad_b])
    batch_t_pad = jnp.concatenate([batch_t, pad_b])

    zeros_d = jnp.zeros((128, D), f32)

    deg_parts = _deg_kernel()(dst_a, dst_t).reshape(2, NW, NPAD)
    dpa = jnp.transpose(deg_parts[0])
    dpt = jnp.transpose(deg_parts[1])

    b1r = b1.reshape(1, D)
    b2r = b2.reshape(1, 2 * D)
    b3r = b3.reshape(1, 4 * D)
    pool_a = _branch(x_a, src_a, dst_a, batch_a_pad, dpa, zeros_d,
                     W1, b1r, W2, b2r, W3, b3r)
    pool_t = _branch(x_t, src_t, dst_t, batch_t_pad, dpt, zeros_d,
                     W1, b1r, W2, b2r, W3, b3r)

    Wo_p = jnp.pad(Wo, ((0, 0), (0, 126)))
    bo_p = jnp.pad(bo, (0, 126)).reshape(1, 128)
    head = _tc_head(pool_a, pool_t, Wg1, bg1.reshape(1, 1024),
                    Wg2, bg2.reshape(1, 128), Wf1, bf1.reshape(1, 1024),
                    Wf2, bf2.reshape(1, 512), Wo_p, bo_p)
    return head[:, :2]


# speedup vs baseline: 14.6273x; 1.5458x over previous
import functools

import jax
import jax.numpy as jnp
from jax import lax
from jax.experimental import pallas as pl
from jax.experimental.pallas import tpu as pltpu
from jax.experimental.pallas import tpu_sc as plsc

f32 = jnp.float32
i32 = jnp.int32

N = 10000
NPAD = 10240
E = 320000
B = 128
D = 128

NC = 2
NS = 16
NW = NC * NS
EPW = E // NW
EC = 80
NCHUNK = EPW // EC
STRIPE = NPAD // NS

RBASE = 312
RPT = 352
PCH = 32
SEGS = 128

@functools.cache
def _mesh():
    return plsc.VectorSubcoreMesh(core_axis_name="c", subcore_axis_name="s",
                                  num_cores=NC, num_subcores=NS)


@functools.cache
def _deg_kernel():
    return pl.kernel(
        _deg_body,
        out_type=jax.ShapeDtypeStruct((2 * NW * NPAD,), f32),
        mesh=_mesh(),
        compiler_params=pltpu.CompilerParams(needs_layout_passes=False),
        scratch_types=[
            pltpu.VMEM((EPW,), i32),
            pltpu.VMEM((NPAD,), f32),
        ],
    )


def _deg_body(dst_a, dst_t, out, dstv, acc):
    c = lax.axis_index("c")
    s = lax.axis_index("s")
    wid = c * NS + s
    for br, dref in ((0, dst_a), (1, dst_t)):
        def zb(i, _):
            acc[pl.ds(i * 16, 16)] = jnp.zeros((16,), f32)
            return 0

        lax.fori_loop(0, NPAD // 16, zb, 0)
        pltpu.sync_copy(dref.at[pl.ds(wid * EPW, EPW)], dstv)

        def body(k, _):
            idx = dstv[pl.ds(k * 16, 16)]
            plsc.addupdate_scatter(acc, [idx], jnp.ones((16,), f32))
            return 0

        lax.fori_loop(0, EPW // 16, body, 0)
        pltpu.sync_copy(acc, out.at[pl.ds((br * NW + wid) * NPAD, NPAD)])


@functools.cache
def _segsum_kernel():
    return pl.kernel(
        _segsum_body,
        out_type=jax.ShapeDtypeStruct((NC, NPAD, D), f32),
        mesh=_mesh(),
        scratch_types=[
            pltpu.VMEM((EC,), i32),
            pltpu.VMEM((EC,), i32),
            pltpu.VMEM((EC,), i32),
            pltpu.VMEM((EC,), i32),
            pltpu.VMEM((EC, D), f32),
            pltpu.VMEM((EC, D), f32),
            pltpu.VMEM((128, D), f32),
            pltpu.VMEM_SHARED((NPAD, D), f32),
            pltpu.SemaphoreType.DMA,
            pltpu.SemaphoreType.DMA,
        ],
    )


def _segsum_body(table, src_hbm, dst_hbm, zeros_d, out, srcv0, srcv1, dstv0,
                 dstv1, rows0, rows1, stage, acc, sem0, sem1):
    c = lax.axis_index("c")
    s = lax.axis_index("s")
    wid = c * NS + s
    pltpu.sync_copy(zeros_d, stage)
    for j in range(STRIPE // 128):
        pltpu.sync_copy(stage, acc.at[pl.ds(s * STRIPE + j * 128, 128)])
    plsc.subcore_barrier()

    eb = wid * EPW

    def load_and_gather(k, srcv, dstv, rows, sem):
        off = eb + k * EC
        pltpu.sync_copy(src_hbm.at[pl.ds(off, EC)], srcv)
        pltpu.sync_copy(dst_hbm.at[pl.ds(off, EC)], dstv)
        return pltpu.async_copy(table.at[srcv], rows, sem)

    load_and_gather(0, srcv0, dstv0, rows0, sem0)

    def pair(i, _):
        k = 2 * i
        load_and_gather(k + 1, srcv1, dstv1, rows1, sem1)
        pltpu.make_async_copy(table.at[srcv0], rows0, sem0).wait()
        pltpu.sync_copy(rows0, acc.at[dstv0], add=True)
        load_and_gather(k + 2, srcv0, dstv0, rows0, sem0)
        pltpu.make_async_copy(table.at[srcv1], rows1, sem1).wait()
        pltpu.sync_copy(rows1, acc.at[dstv1], add=True)
        return 0

    lax.fori_loop(0, (NCHUNK - 1) // 2, pair, 0)
    pltpu.make_async_copy(table.at[srcv0], rows0, sem0).wait()
    pltpu.sync_copy(rows0, acc.at[dstv0], add=True)
    plsc.subcore_barrier()
    for j in range(STRIPE // 128):
        pltpu.sync_copy(acc.at[pl.ds(s * STRIPE + j * 128, 128)], stage)

        @pl.when(c == 0)
        def _():
            pltpu.sync_copy(stage, out.at[0, pl.ds(s * STRIPE + j * 128, 128)])

        @pl.when(c == 1)
        def _():
            pltpu.sync_copy(stage, out.at[1, pl.ds(s * STRIPE + j * 128, 128)])


@functools.cache
def _pool_kernel():
    return pl.kernel(
        _pool_body,
        out_type=jax.ShapeDtypeStruct((NW * SEGS * 512,), f32),
        mesh=_mesh(),
        compiler_params=pltpu.CompilerParams(needs_layout_passes=False),
        scratch_types=[
            pltpu.VMEM((RPT,), i32),
            pltpu.VMEM((PCH * 512,), f32),
            pltpu.VMEM((129 * 512,), f32),
        ],
    )


def _pool_body(h3_flat, batch_pad, out, bv, rbuf, acc):
    c = lax.axis_index("c")
    s = lax.axis_index("s")
    wid = c * NS + s
    base = wid * RBASE

    def zb(i, _):
        acc[pl.ds(i * 16, 16)] = jnp.full((16,), -1.0, f32)
        return 0

    lax.fori_loop(0, (129 * 512) // 16, zb, 0)
    pltpu.sync_copy(batch_pad.at[pl.ds(base, RPT)], bv)
    lane = lax.iota(i32, 16)

    def chunk(j, _):
        pltpu.sync_copy(h3_flat.at[pl.ds((base + j * PCH) * 512, PCH * 512)],
                        rbuf)

        def row(l, _):
            bidx = jnp.full((16,), 0, i32) + (j * PCH + l)
            b16 = plsc.load_gather(bv, [bidx])
            addr0 = b16 * 512 + lane
            roff = l * 512
            for q in range(32):
                addr = addr0 + q * 16
                a = plsc.load_gather(acc, [addr])
                r = rbuf[pl.ds(roff + q * 16, 16)]
                plsc.store_scatter(acc, [addr], jnp.maximum(a, r))
            return 0

        lax.fori_loop(0, PCH, row, 0)
        return 0

    lax.fori_loop(0, RPT // PCH, chunk, 0)
    pltpu.sync_copy(acc.at[pl.ds(0, SEGS * 512)],
                    out.at[pl.ds(wid * SEGS * 512, SEGS * 512)])


BLK = 1024


def _tc_pre(x_pad, deg_partsT):
    def body(x_ref, d_ref, xp_ref, dinv_ref):
        deg = jnp.sum(d_ref[...], axis=1, keepdims=True)
        dinv = lax.rsqrt(deg + 1.0)
        xp_ref[...] = x_ref[...] * dinv
        dinv_ref[...] = dinv

    return pl.pallas_call(
        body,
        grid=(NPAD // BLK,),
        in_specs=[pl.BlockSpec((BLK, D), lambda i: (i, 0)),
                  pl.BlockSpec((BLK, NW), lambda i: (i, 0))],
        out_specs=[pl.BlockSpec((BLK, D), lambda i: (i, 0)),
                   pl.BlockSpec((BLK, 1), lambda i: (i, 0))],
        out_shape=[jax.ShapeDtypeStruct((NPAD, D), f32),
                   jax.ShapeDtypeStruct((NPAD, 1), f32)],
    )(x_pad, deg_partsT)


def _tc_layer1(parts, xp, dinv, W, b):
    def body(p_ref, x_ref, v_ref, w_ref, b_ref, o_ref):
        p = p_ref[...]
        dv = v_ref[...]
        t = (p[0] + p[1] + x_ref[...]) * dv
        h = jnp.dot(t, w_ref[...], preferred_element_type=f32) + b_ref[...]
        o_ref[...] = jnp.maximum(h, 0.0) * dv

    return pl.pallas_call(
        body,
        grid=(NPAD // BLK,),
        in_specs=[pl.BlockSpec((NC, BLK, D), lambda i: (0, i, 0)),
                  pl.BlockSpec((BLK, D), lambda i: (i, 0)),
                  pl.BlockSpec((BLK, 1), lambda i: (i, 0)),
                  pl.BlockSpec((D, D), lambda i: (0, 0)),
                  pl.BlockSpec((1, D), lambda i: (0, 0))],
        out_specs=pl.BlockSpec((BLK, D), lambda i: (i, 0)),
        out_shape=jax.ShapeDtypeStruct((NPAD, D), f32),
    )(parts, xp, dinv, W, b)


def _tc_layer2(parts, h1p, dinv, W, b):
    def body(p_ref, x_ref, v_ref, w_ref, b_ref, o0_ref, o1_ref):
        p = p_ref[...]
        dv = v_ref[...]
        t = (p[0] + p[1] + x_ref[...]) * dv
        h = jnp.dot(t, w_ref[...], preferred_element_type=f32) + b_ref[...]
        hp = jnp.maximum(h, 0.0) * dv
        o0_ref[...] = hp[:, :D]
        o1_ref[...] = hp[:, D:]

    return pl.pallas_call(
        body,
        grid=(NPAD // BLK,),
        in_specs=[pl.BlockSpec((NC, BLK, D), lambda i: (0, i, 0)),
                  pl.BlockSpec((BLK, D), lambda i: (i, 0)),
                  pl.BlockSpec((BLK, 1), lambda i: (i, 0)),
                  pl.BlockSpec((D, 2 * D), lambda i: (0, 0)),
                  pl.BlockSpec((1, 2 * D), lambda i: (0, 0))],
        out_specs=[pl.BlockSpec((BLK, D), lambda i: (i, 0)),
                   pl.BlockSpec((BLK, D), lambda i: (i, 0))],
        out_shape=[jax.ShapeDtypeStruct((NPAD, D), f32),
                   jax.ShapeDtypeStruct((NPAD, D), f32)],
    )(parts, h1p, dinv, W, b)


def _tc_layer3(pa, pb, h2p0, h2p1, dinv, W, b):
    def body(pa_ref, pb_ref, x0_ref, x1_ref, v_ref, w_ref, b_ref, o_ref):
        pa_ = pa_ref[...]
        pb_ = pb_ref[...]
        dv = v_ref[...]
        t0 = (pa_[0] + pa_[1] + x0_ref[...]) * dv
        t1 = (pb_[0] + pb_[1] + x1_ref[...]) * dv
        t = jnp.concatenate([t0, t1], axis=1)
        h = jnp.dot(t, w_ref[...], preferred_element_type=f32) + b_ref[...]
        o_ref[...] = jnp.maximum(h, 0.0)

    return pl.pallas_call(
        body,
        grid=(NPAD // BLK,),
        in_specs=[pl.BlockSpec((NC, BLK, D), lambda i: (0, i, 0)),
                  pl.BlockSpec((NC, BLK, D), lambda i: (0, i, 0)),
                  pl.BlockSpec((BLK, D), lambda i: (i, 0)),
                  pl.BlockSpec((BLK, D), lambda i: (i, 0)),
                  pl.BlockSpec((BLK, 1), lambda i: (i, 0)),
                  pl.BlockSpec((2 * D, 4 * D), lambda i: (0, 0)),
                  pl.BlockSpec((1, 4 * D), lambda i: (0, 0))],
        out_specs=pl.BlockSpec((BLK, 4 * D), lambda i: (i, 0)),
        out_shape=jax.ShapeDtypeStruct((NPAD, 4 * D), f32),
    )(pa, pb, h2p0, h2p1, dinv, W, b)


def _tc_head(pa, pt, Wg1, bg1, Wg2, bg2, Wf1, bf1, Wf2, bf2, Wo_p, bo_p):
    def body(pa_ref, pt_ref, wg1, bg1r, wg2, bg2r, wf1, bf1r, wf2, bf2r, wo,
             bor, o_ref):
        def gbranch(pref):
            pm = jnp.max(pref[...], axis=0)
            pm = jnp.where(pm < 0.0, 0.0, pm)
            g = jnp.dot(pm, wg1[...], preferred_element_type=f32) + bg1r[...]
            g = jnp.maximum(g, 0.0)
            return jnp.dot(g, wg2[...], preferred_element_type=f32) + bg2r[...]

        ga = gbranch(pa_ref)
        gt = gbranch(pt_ref)
        xc = jnp.concatenate([ga, gt], axis=1)
        y = jnp.dot(xc, wf1[...], preferred_element_type=f32) + bf1r[...]
        y = jnp.maximum(y, 0.0)
        y = jnp.dot(y, wf2[...], preferred_element_type=f32) + bf2r[...]
        y = jnp.maximum(y, 0.0)
        o_ref[...] = jnp.dot(y, wo[...], preferred_element_type=f32) + bor[...]

    return pl.pallas_call(
        body,
        out_shape=jax.ShapeDtypeStruct((B, 128), f32),
    )(pa, pt, Wg1, bg1, Wg2, bg2, Wf1, bf1, Wf2, bf2, Wo_p, bo_p)


def _branch(x, src, dst, batch_pad, deg_partsT, zeros_d, W1, b1, W2, b2, W3,
            b3):
    x_pad = jnp.pad(x, ((0, NPAD - N), (0, 0)))
    xp, dinv = _tc_pre(x_pad, deg_partsT)
    p1 = _segsum_kernel()(xp, src, dst, zeros_d)
    h1p = _tc_layer1(p1, xp, dinv, W1, b1)
    p2 = _segsum_kernel()(h1p, src, dst, zeros_d)
    h2p0, h2p1 = _tc_layer2(p2, h1p, dinv, W2, b2)
    pa = _segsum_kernel()(h2p0, src, dst, zeros_d)
    pb = _segsum_kernel()(h2p1, src, dst, zeros_d)
    h3 = _tc_layer3(pa, pb, h2p0, h2p1, dinv, W3, b3)
    pool = _pool_kernel()(h3.reshape(-1), batch_pad)
    return pool.reshape(NW, SEGS, 512)


def kernel(x_a, edge_index_a, batch_a, x_t, edge_index_t, batch_t,
           W1, b1, W2, b2, W3, b3, Wg1, bg1, Wg2, bg2,
           Wf1, bf1, Wf2, bf2, Wo, bo):
    src_a = edge_index_a[0]
    dst_a = edge_index_a[1]
    src_t = edge_index_t[0]
    dst_t = edge_index_t[1]
    pad_b = jnp.full((NPAD - N,), SEGS, i32)
    batch_a_pad = jnp.concatenate([batch_a, pad_b])
    batch_t_pad = jnp.concatenate([batch_t, pad_b])

    zeros_d = jnp.zeros((128, D), f32)

    deg_parts = _deg_kernel()(dst_a, dst_t).reshape(2, NW, NPAD)
    dpa = jnp.transpose(deg_parts[0])
    dpt = jnp.transpose(deg_parts[1])

    b1r = b1.reshape(1, D)
    b2r = b2.reshape(1, 2 * D)
    b3r = b3.reshape(1, 4 * D)
    pool_a = _branch(x_a, src_a, dst_a, batch_a_pad, dpa, zeros_d,
                     W1, b1r, W2, b2r, W3, b3r)
    pool_t = _branch(x_t, src_t, dst_t, batch_t_pad, dpt, zeros_d,
                     W1, b1r, W2, b2r, W3, b3r)

    Wo_p = jnp.pad(Wo, ((0, 0), (0, 126)))
    bo_p = jnp.pad(bo, (0, 126)).reshape(1, 128)
    head = _tc_head(pool_a, pool_t, Wg1, bg1.reshape(1, 1024),
                    Wg2, bg2.reshape(1, 128), Wf1, bf1.reshape(1, 1024),
                    Wf2, bf2.reshape(1, 512), Wo_p, bo_p)
    return head[:, :2]
